# baseline (speedup 1.0000x reference)
import jax
import jax.numpy as jnp
from jax.experimental import pallas as pl
from jax.experimental.pallas import tpu as pltpu

_R = 32


def _body(embt_ref, out_ref, bt_ref):
    Sq = pl.num_programs(0) * _R
    Sv = out_ref.shape[2]
    d, n_emb = embt_ref.shape
    max_pos = (n_emb - 1) // 2
    lo = Sq - max_pos
    hi = Sq + max_pos

    p = pl.program_id(0)

    @pl.when(p == 0)
    def _():
        e = embt_ref[...]
        bt_ref[:, 0:lo] = jnp.broadcast_to(e[:, 0:1], (d, lo))
        bt_ref[:, lo:hi] = e[:, 1:n_emb]
        bt_ref[:, hi:] = jnp.broadcast_to(
            e[:, n_emb - 1 : n_emb], (d, bt_ref.shape[1] - hi)
        )

    i0 = p * _R
    t = (Sq - _R) - i0
    rot = pltpu.roll(bt_ref[...], -t, axis=1)
    for r in range(_R):
        off = (_R - 1) - r
        out_ref[r, :, :] = rot[:, off : off + Sv]


def kernel(q, v, embeddings):
    Sq = q.shape[1]
    Sv = v.shape[1]
    n_emb, d = embeddings.shape
    out = pl.pallas_call(
        _body,
        grid=(Sq // _R,),
        in_specs=[pl.BlockSpec((d, n_emb), lambda p: (0, 0))],
        out_specs=pl.BlockSpec((_R, d, Sv), lambda p: (p, 0, 0)),
        out_shape=jax.ShapeDtypeStruct((Sq, d, Sv), embeddings.dtype),
        scratch_shapes=[pltpu.VMEM((d, Sq + Sv), embeddings.dtype)],
    )(embeddings.T)
    return out.transpose(0, 2, 1)
